# Initial kernel scaffold; baseline (speedup 1.0000x reference)
import jax
import jax.numpy as jnp
from jax.experimental import pallas as pl


def kernel(xq, xt, Wm, bm, W1, a_src1, a_dst1, b1, W2, a_src2, a_dst2, b2, W3, a_src3, a_dst3, b3, edge_index_q, edge_index_t):
    raise NotImplementedError("write your pallas kernel here")


# speedup vs baseline: 9.9264x; 9.9264x over previous
import functools

import jax
import jax.numpy as jnp
from jax import lax
from jax.experimental import pallas as pl
from jax.experimental.pallas import tpu as pltpu
from jax.experimental.pallas import tpu_sc as plsc

N = 10000
NP = 10240
D = 128
E = 320000
E2 = E + N
NW = 32
ET = 10320
EP = NW * ET
CHUNKS = ET // 16
ROWS_PER_SUB = NP // 16

_MESH = dict(core_axis_name="c", subcore_axis_name="s",
             num_cores=2, num_subcores=16)


def _t1a_body(xq_ref, xt_ref, wm_ref, bm_ref, w_ref, a2_ref,
              hq_ref, ht_ref, uvq_ref, uvt_ref):
    wm = wm_ref[...]
    bm = bm_ref[...]
    w = w_ref[...]
    a2 = a2_ref[...]
    for x_ref, h_ref, uv_ref in ((xq_ref, hq_ref, uvq_ref),
                                 (xt_ref, ht_ref, uvt_ref)):
        x = jnp.dot(x_ref[...], wm, preferred_element_type=jnp.float32, precision=lax.Precision.HIGHEST) + bm
        h = jnp.dot(x, w, preferred_element_type=jnp.float32, precision=lax.Precision.HIGHEST)
        h_ref[...] = h
        uv_ref[...] = jnp.dot(h, a2, preferred_element_type=jnp.float32, precision=lax.Precision.HIGHEST)


def _elu(x):
    return jnp.where(x > 0, x, jnp.exp(jnp.minimum(x, 0.0)) - 1.0)


def _t1b_body(pq_ref, pt_ref, bprev_ref, w_ref, a2_ref,
              hq_ref, ht_ref, uvq_ref, uvt_ref):
    bprev = bprev_ref[...]
    w = w_ref[...]
    a2 = a2_ref[...]
    for p_ref, h_ref, uv_ref in ((pq_ref, hq_ref, uvq_ref),
                                 (pt_ref, ht_ref, uvt_ref)):
        x = _elu(p_ref[...] + bprev)
        h = jnp.dot(x, w, preferred_element_type=jnp.float32, precision=lax.Precision.HIGHEST)
        h_ref[...] = h
        uv_ref[...] = jnp.dot(h, a2, preferred_element_type=jnp.float32, precision=lax.Precision.HIGHEST)


def _t3_body(pq_ref, pt_ref, b_ref, xq_ref, xt_ref):
    b = b_ref[...]
    xq_ref[...] = _elu(pq_ref[...] + b)
    xt_ref[...] = _elu(pt_ref[...] + b)


def _t2_body(dq_ref, dt_ref, rq_ref, rt_ref):
    rq_ref[...] = 1.0 / (jnp.sum(dq_ref[...], axis=0, keepdims=True) + 1e-16)
    rt_ref[...] = 1.0 / (jnp.sum(dt_ref[...], axis=0, keepdims=True) + 1e-16)


_BLK = 256
_GRID = NP // _BLK


def _row_spec(d):
    return pl.BlockSpec((_BLK, d), lambda i: (i, 0))


def _full_spec(shape):
    return pl.BlockSpec(shape, lambda i: tuple(0 for _ in shape))


_t1a = pl.pallas_call(
    _t1a_body,
    grid=(_GRID,),
    in_specs=[_row_spec(D), _row_spec(D), _full_spec((D, D)),
              _full_spec((1, D)), _full_spec((D, D)), _full_spec((D, 4))],
    out_specs=[_row_spec(D), _row_spec(D), _row_spec(4), _row_spec(4)],
    out_shape=[jax.ShapeDtypeStruct((NP, D), jnp.float32),
               jax.ShapeDtypeStruct((NP, D), jnp.float32),
               jax.ShapeDtypeStruct((NP, 4), jnp.float32),
               jax.ShapeDtypeStruct((NP, 4), jnp.float32)],
)

_t1b = pl.pallas_call(
    _t1b_body,
    grid=(_GRID,),
    in_specs=[_row_spec(D), _row_spec(D), _full_spec((1, D)),
              _full_spec((D, D)), _full_spec((D, 4))],
    out_specs=[_row_spec(D), _row_spec(D), _row_spec(4), _row_spec(4)],
    out_shape=[jax.ShapeDtypeStruct((NP, D), jnp.float32),
               jax.ShapeDtypeStruct((NP, D), jnp.float32),
               jax.ShapeDtypeStruct((NP, 4), jnp.float32),
               jax.ShapeDtypeStruct((NP, 4), jnp.float32)],
)

_t3 = pl.pallas_call(
    _t3_body,
    grid=(_GRID,),
    in_specs=[_row_spec(D), _row_spec(D), _full_spec((1, D))],
    out_specs=[_row_spec(D), _row_spec(D)],
    out_shape=[jax.ShapeDtypeStruct((NP, D), jnp.float32),
               jax.ShapeDtypeStruct((NP, D), jnp.float32)],
)

_t2 = pl.pallas_call(
    _t2_body,
    in_specs=[pl.BlockSpec(memory_space=pltpu.VMEM),
              pl.BlockSpec(memory_space=pltpu.VMEM)],
    out_specs=[pl.BlockSpec(memory_space=pltpu.VMEM),
               pl.BlockSpec(memory_space=pltpu.VMEM)],
    out_shape=[jax.ShapeDtypeStruct((1, NP), jnp.float32),
               jax.ShapeDtypeStruct((1, NP), jnp.float32)],
)


def _s1_body(uv_hbm, s_hbm, d_hbm, ex_hbm, den_hbm,
             uv_v, s_v, d_v, ex_v, den_v, tmp_d, tmp_c):
    cid = lax.axis_index("c")
    sid = lax.axis_index("s")
    wid = sid * 2 + cid
    base = wid * ET
    pltpu.sync_copy(uv_hbm, uv_v)
    pltpu.sync_copy(s_hbm.at[pl.ds(base, ET)], s_v)
    pltpu.sync_copy(d_hbm.at[pl.ds(base, ET)], d_v)

    zf = jnp.zeros((16,), jnp.float32)

    def zero_body(i, c):
        den_v[pl.ds(i * 16, 16)] = zf
        return c

    lax.fori_loop(0, NP // 16, zero_body, 0)

    col0 = jnp.zeros((16,), jnp.int32)
    col1 = jnp.ones((16,), jnp.int32)
    iota = lax.iota(jnp.int32, 16)
    prev_i = jnp.maximum(iota - 1, 0)
    next_i = jnp.minimum(iota + 1, 15)
    first = iota == 0
    last = iota == 15

    def edge_body(i, c):
        off = i * 16
        s16 = s_v[pl.ds(off, 16)]
        d16 = d_v[pl.ds(off, 16)]
        u = plsc.load_gather(uv_v, [s16, col0])
        v = plsc.load_gather(uv_v, [d16, col1])
        e = u + v
        e = jnp.maximum(e, e * 0.2)
        ex = jnp.exp(e)
        ex_v[pl.ds(off, 16)] = ex
        dk, exs = plsc.sort_key_val(d16, ex)
        tmp_d[...] = dk
        c1 = plsc.cumsum(exs)
        tmp_c[...] = c1
        dprev = plsc.load_gather(tmp_d, [prev_i])
        dnext = plsc.load_gather(tmp_d, [next_i])
        cprev = jnp.where(first, 0.0, plsc.load_gather(tmp_c, [prev_i]))
        run_start = (dk != dprev) | first
        run_end = (dk != dnext) | last
        base = plsc.cummax(jnp.where(run_start, cprev, 0.0))
        plsc.addupdate_scatter(den_v, [dk], c1 - base, mask=run_end)
        return c

    lax.fori_loop(0, CHUNKS, edge_body, 0)

    pltpu.sync_copy(ex_v, ex_hbm.at[pl.ds(base, ET)])
    pltpu.sync_copy(den_v, den_hbm.at[wid])


_SC_PARAMS = pltpu.CompilerParams(needs_layout_passes=False,
                                  use_tc_tiling_on_sc=False)

_s1 = functools.partial(
    pl.kernel,
    out_type=[jax.ShapeDtypeStruct((EP,), jnp.float32),
              jax.ShapeDtypeStruct((NW, NP), jnp.float32)],
    compiler_params=_SC_PARAMS,
    scratch_types=[pltpu.VMEM((NP, 4), jnp.float32),
                   pltpu.VMEM((ET,), jnp.int32),
                   pltpu.VMEM((ET,), jnp.int32),
                   pltpu.VMEM((ET,), jnp.float32),
                   pltpu.VMEM((NP,), jnp.float32),
                   pltpu.VMEM((16,), jnp.int32),
                   pltpu.VMEM((16,), jnp.float32)],
)


ROWS_PER_W = NP // NW
BUF = 6144
EPP = EP + BUF


def _s2_body(h_hbm, rden_hbm, s_hbm, d_hbm, ex_hbm, bounds_hbm, out_hbm,
             rden_v, s_v, d_v, ex_v, rows_v, alpha_b, sidx_b, dloc_b,
             out_local, bounds_v, sem):
    cid = lax.axis_index("c")
    sid = lax.axis_index("s")
    wid = sid * 2 + cid
    row0 = wid * ROWS_PER_W

    pltpu.sync_copy(rden_hbm, rden_v)
    pltpu.sync_copy(bounds_hbm, bounds_v)

    zf = jnp.zeros((16,), jnp.float32)

    def zero_body(i, c):
        out_local[pl.ds(i * 16, 16)] = zf
        return c

    lax.fori_loop(0, ROWS_PER_W * D // 16, zero_body, 0)

    lo = plsc.load_gather(bounds_v, [jnp.full((16,), wid, jnp.int32)])[0]
    hi = plsc.load_gather(bounds_v, [jnp.full((16,), wid + 1, jnp.int32)])[0]
    st8 = lo & ~7
    nseg = (hi - st8 + (BUF - 1)) // BUF

    col0 = jnp.zeros((16,), jnp.int32)
    iota = lax.iota(jnp.int32, 16)

    def seg_body(g, c):
        start = pl.multiple_of(st8 + g * BUF, 8)
        pltpu.sync_copy(s_hbm.at[pl.ds(start, BUF)], s_v)
        pltpu.sync_copy(d_hbm.at[pl.ds(start, BUF)], d_v)
        pltpu.sync_copy(ex_hbm.at[pl.ds(start, BUF)], ex_v)
        nch = (jnp.minimum(hi, start + BUF) - start + 15) // 16

        def edge_body(i, c2):
            off = i * 16
            gidx = start + off + iota
            s16 = s_v[pl.ds(off, 16)]
            d16 = d_v[pl.ds(off, 16)]
            ex16 = ex_v[pl.ds(off, 16)]
            valid = (gidx >= lo) & (gidx < hi)
            r16 = plsc.load_gather(rden_v, [col0, d16])
            alpha16 = jnp.where(valid, ex16 * r16, 0.0)
            dloc16 = jnp.where(valid, d16 - row0, 0) * D
            pltpu.async_copy(h_hbm.at[s16], rows_v, sem).wait()
            for k in range(16):
                av = jnp.full((16,), alpha16[k], jnp.float32)
                base16 = jnp.full((16,), dloc16[k], jnp.int32) + iota
                for j in range(8):
                    idx = base16 + j * 16
                    plsc.addupdate_scatter(
                        out_local, [idx], av * rows_v[k, pl.ds(j * 16, 16)])
            return c2

        lax.fori_loop(0, nch, edge_body, 0)
        return c

    lax.fori_loop(0, nseg, seg_body, 0)
    pltpu.sync_copy(out_local, out_hbm.at[pl.ds(row0 * D, ROWS_PER_W * D)])


_s2 = functools.partial(
    pl.kernel,
    out_type=jax.ShapeDtypeStruct((NP * D,), jnp.float32),
    compiler_params=_SC_PARAMS,
    scratch_types=[pltpu.VMEM((1, NP), jnp.float32),
                   pltpu.VMEM((BUF,), jnp.int32),
                   pltpu.VMEM((BUF,), jnp.int32),
                   pltpu.VMEM((BUF,), jnp.float32),
                   pltpu.VMEM((16, D), jnp.float32),
                   pltpu.VMEM((16,), jnp.float32),
                   pltpu.VMEM((16,), jnp.int32),
                   pltpu.VMEM((16,), jnp.int32),
                   pltpu.VMEM((ROWS_PER_W * D,), jnp.float32),
                   pltpu.VMEM((48,), jnp.int32),
                   pltpu.SemaphoreType.DMA],
)


def kernel(xq, xt, Wm, bm, W1, a_src1, a_dst1, b1, W2, a_src2, a_dst2, b2,
           W3, a_src3, a_dst3, b3, edge_index_q, edge_index_t):
    del edge_index_t
    f32 = jnp.float32
    src = edge_index_q[0].astype(jnp.int32)
    dst = edge_index_q[1].astype(jnp.int32)
    loop = jnp.arange(N, dtype=jnp.int32)
    padv = jnp.full((EP - E2,), N, jnp.int32)
    s2 = jnp.concatenate([src, loop, padv])
    d2 = jnp.concatenate([dst, loop, padv])
    perm = jnp.argsort(d2)
    d2 = d2[perm]
    s2 = s2[perm]
    bounds = jnp.searchsorted(
        d2, jnp.arange(NW + 1, dtype=jnp.int32) * ROWS_PER_W
    ).astype(jnp.int32)
    bounds = jnp.concatenate(
        [bounds, jnp.full((48 - NW - 1,), EP, jnp.int32)])
    s2p = jnp.concatenate([s2, jnp.zeros((BUF,), jnp.int32)])
    d2p = jnp.concatenate([d2, jnp.full((BUF,), N, jnp.int32)])

    xq_p = jnp.zeros((NP, D), f32).at[:N].set(xq.astype(f32))
    xt_p = jnp.zeros((NP, D), f32).at[:N].set(xt.astype(f32))
    bm2 = bm.reshape(1, D).astype(f32)

    def a2_of(a_s, a_d):
        z = jnp.zeros((D, 1), f32)
        return jnp.concatenate(
            [a_s.reshape(D, 1), a_d.reshape(D, 1), z, z], axis=1)

    layers = [(W1.astype(f32), a2_of(a_src1, a_dst1), b1.reshape(1, D)),
              (W2.astype(f32), a2_of(a_src2, a_dst2), b2.reshape(1, D)),
              (W3.astype(f32), a2_of(a_src3, a_dst3), b3.reshape(1, D))]

    mesh = plsc.VectorSubcoreMesh(**_MESH)
    s1 = _s1(_s1_body, mesh=mesh)
    s2k = _s2(_s2_body, mesh=mesh)

    pq = pt = None
    for li, (W, a2, b) in enumerate(layers):
        if li == 0:
            hq, ht, uvq, uvt = _t1a(xq_p, xt_p, Wm.astype(f32), bm2, W, a2)
        else:
            hq, ht, uvq, uvt = _t1b(pq, pt, layers[li - 1][2], W, a2)
        exq, denq = s1(uvq, s2, d2)
        dep = (denq[0, 0] * 0.0).astype(jnp.float32)
        ext, dent = s1(uvt + dep, s2, d2)
        rq, rt = _t2(denq, dent)
        zpad = jnp.zeros((BUF,), f32)
        exq_p = jnp.concatenate([exq, zpad])
        ext_p = jnp.concatenate([ext, zpad])
        pq = s2k(hq, rq + (rt[0, 0] * 0.0), s2p, d2p, exq_p, bounds)
        pt = s2k(ht, rt + (pq[0] * 0.0), s2p, d2p, ext_p, bounds)
        pq = pq.reshape(NP, D)
        pt = pt.reshape(NP, D)
    Xq, Xt = _t3(pq, pt, layers[2][2])
    return Xq[:N], Xt[:N]


# speedup vs baseline: 10.5104x; 1.0588x over previous
import functools

import jax
import jax.numpy as jnp
from jax import lax
from jax.experimental import pallas as pl
from jax.experimental.pallas import tpu as pltpu
from jax.experimental.pallas import tpu_sc as plsc

N = 10000
NP = 10240
D = 128
E = 320000
E2 = E + N
NW = 32
ET = 10320
EP = NW * ET
CHUNKS = ET // 16
ROWS_PER_SUB = NP // 16

_MESH = dict(core_axis_name="c", subcore_axis_name="s",
             num_cores=2, num_subcores=16)


def _t1a_body(xq_ref, xt_ref, wm_ref, bm_ref, w_ref, a2_ref,
              hq_ref, ht_ref, uvq_ref, uvt_ref):
    wm = wm_ref[...]
    bm = bm_ref[...]
    w = w_ref[...]
    a2 = a2_ref[...]
    for x_ref, h_ref, uv_ref in ((xq_ref, hq_ref, uvq_ref),
                                 (xt_ref, ht_ref, uvt_ref)):
        x = jnp.dot(x_ref[...], wm, preferred_element_type=jnp.float32, precision=lax.Precision.HIGHEST) + bm
        h = jnp.dot(x, w, preferred_element_type=jnp.float32, precision=lax.Precision.HIGHEST)
        h_ref[...] = h
        uv_ref[...] = jnp.dot(h, a2, preferred_element_type=jnp.float32, precision=lax.Precision.HIGHEST)


def _elu(x):
    return jnp.where(x > 0, x, jnp.exp(jnp.minimum(x, 0.0)) - 1.0)


def _t1b_body(pq_ref, pt_ref, bprev_ref, w_ref, a2_ref,
              hq_ref, ht_ref, uvq_ref, uvt_ref):
    bprev = bprev_ref[...]
    w = w_ref[...]
    a2 = a2_ref[...]
    for p_ref, h_ref, uv_ref in ((pq_ref, hq_ref, uvq_ref),
                                 (pt_ref, ht_ref, uvt_ref)):
        x = _elu(p_ref[...] + bprev)
        h = jnp.dot(x, w, preferred_element_type=jnp.float32, precision=lax.Precision.HIGHEST)
        h_ref[...] = h
        uv_ref[...] = jnp.dot(h, a2, preferred_element_type=jnp.float32, precision=lax.Precision.HIGHEST)


def _t3_body(pq_ref, pt_ref, b_ref, xq_ref, xt_ref):
    b = b_ref[...]
    xq_ref[...] = _elu(pq_ref[...] + b)
    xt_ref[...] = _elu(pt_ref[...] + b)


def _t2_body(dq_ref, dt_ref, rq_ref, rt_ref):
    rq_ref[...] = 1.0 / (jnp.sum(dq_ref[...], axis=0, keepdims=True) + 1e-16)
    rt_ref[...] = 1.0 / (jnp.sum(dt_ref[...], axis=0, keepdims=True) + 1e-16)


_BLK = 256
_GRID = NP // _BLK


def _row_spec(d):
    return pl.BlockSpec((_BLK, d), lambda i: (i, 0))


def _full_spec(shape):
    return pl.BlockSpec(shape, lambda i: tuple(0 for _ in shape))


_t1a = pl.pallas_call(
    _t1a_body,
    grid=(_GRID,),
    in_specs=[_row_spec(D), _row_spec(D), _full_spec((D, D)),
              _full_spec((1, D)), _full_spec((D, D)), _full_spec((D, 4))],
    out_specs=[_row_spec(D), _row_spec(D), _row_spec(4), _row_spec(4)],
    out_shape=[jax.ShapeDtypeStruct((NP, D), jnp.float32),
               jax.ShapeDtypeStruct((NP, D), jnp.float32),
               jax.ShapeDtypeStruct((NP, 4), jnp.float32),
               jax.ShapeDtypeStruct((NP, 4), jnp.float32)],
)

_t1b = pl.pallas_call(
    _t1b_body,
    grid=(_GRID,),
    in_specs=[_row_spec(D), _row_spec(D), _full_spec((1, D)),
              _full_spec((D, D)), _full_spec((D, 4))],
    out_specs=[_row_spec(D), _row_spec(D), _row_spec(4), _row_spec(4)],
    out_shape=[jax.ShapeDtypeStruct((NP, D), jnp.float32),
               jax.ShapeDtypeStruct((NP, D), jnp.float32),
               jax.ShapeDtypeStruct((NP, 4), jnp.float32),
               jax.ShapeDtypeStruct((NP, 4), jnp.float32)],
)

_t3 = pl.pallas_call(
    _t3_body,
    grid=(_GRID,),
    in_specs=[_row_spec(D), _row_spec(D), _full_spec((1, D))],
    out_specs=[_row_spec(D), _row_spec(D)],
    out_shape=[jax.ShapeDtypeStruct((NP, D), jnp.float32),
               jax.ShapeDtypeStruct((NP, D), jnp.float32)],
)

_t2 = pl.pallas_call(
    _t2_body,
    in_specs=[pl.BlockSpec(memory_space=pltpu.VMEM),
              pl.BlockSpec(memory_space=pltpu.VMEM)],
    out_specs=[pl.BlockSpec(memory_space=pltpu.VMEM),
               pl.BlockSpec(memory_space=pltpu.VMEM)],
    out_shape=[jax.ShapeDtypeStruct((1, NP), jnp.float32),
               jax.ShapeDtypeStruct((1, NP), jnp.float32)],
)


def _s1_body(uv_hbm, s_hbm, d_hbm, ex_hbm, den_hbm,
             uv_v, s_v, d_v, ex_v, den_v, tmp_d, tmp_c):
    cid = lax.axis_index("c")
    sid = lax.axis_index("s")
    wid = sid * 2 + cid
    base = wid * ET
    pltpu.sync_copy(uv_hbm, uv_v)
    pltpu.sync_copy(s_hbm.at[pl.ds(base, ET)], s_v)
    pltpu.sync_copy(d_hbm.at[pl.ds(base, ET)], d_v)

    zf = jnp.zeros((16,), jnp.float32)

    def zero_body(i, c):
        den_v[pl.ds(i * 16, 16)] = zf
        return c

    lax.fori_loop(0, NP // 16, zero_body, 0)

    col0 = jnp.zeros((16,), jnp.int32)
    col1 = jnp.ones((16,), jnp.int32)
    iota = lax.iota(jnp.int32, 16)
    prev_i = jnp.maximum(iota - 1, 0)
    next_i = jnp.minimum(iota + 1, 15)
    first = iota == 0
    last = iota == 15

    def edge_body(i, c):
        off = i * 16
        s16 = s_v[pl.ds(off, 16)]
        d16 = d_v[pl.ds(off, 16)]
        u = plsc.load_gather(uv_v, [s16, col0])
        v = plsc.load_gather(uv_v, [d16, col1])
        e = u + v
        e = jnp.maximum(e, e * 0.2)
        ex = jnp.exp(e)
        ex_v[pl.ds(off, 16)] = ex
        dk, exs = plsc.sort_key_val(d16, ex)
        tmp_d[...] = dk
        c1 = plsc.cumsum(exs)
        tmp_c[...] = c1
        dprev = plsc.load_gather(tmp_d, [prev_i])
        dnext = plsc.load_gather(tmp_d, [next_i])
        cprev = jnp.where(first, 0.0, plsc.load_gather(tmp_c, [prev_i]))
        run_start = (dk != dprev) | first
        run_end = (dk != dnext) | last
        base = plsc.cummax(jnp.where(run_start, cprev, 0.0))
        plsc.addupdate_scatter(den_v, [dk], c1 - base, mask=run_end)
        return c

    lax.fori_loop(0, CHUNKS, edge_body, 0)

    pltpu.sync_copy(ex_v, ex_hbm.at[pl.ds(base, ET)])
    pltpu.sync_copy(den_v, den_hbm.at[wid])


_SC_PARAMS = pltpu.CompilerParams(needs_layout_passes=False,
                                  use_tc_tiling_on_sc=False)

_s1 = functools.partial(
    pl.kernel,
    out_type=[jax.ShapeDtypeStruct((EP,), jnp.float32),
              jax.ShapeDtypeStruct((NW, NP), jnp.float32)],
    compiler_params=_SC_PARAMS,
    scratch_types=[pltpu.VMEM((NP, 4), jnp.float32),
                   pltpu.VMEM((ET,), jnp.int32),
                   pltpu.VMEM((ET,), jnp.int32),
                   pltpu.VMEM((ET,), jnp.float32),
                   pltpu.VMEM((NP,), jnp.float32),
                   pltpu.VMEM((16,), jnp.int32),
                   pltpu.VMEM((16,), jnp.float32)],
)


ROWS_PER_W = NP // NW
BUF = 6144
EPP = EP + BUF


def _s2_body(h_hbm, rden_hbm, s_hbm, d_hbm, ex_hbm, bounds_hbm, out_hbm,
             rden_v, s_v, d_v, ex_v, r0, r1, r2, r3,
             out_local, bounds_v, m0, m1, m2, m3):
    rows4 = (r0, r1, r2, r3)
    sems = (m0, m1, m2, m3)
    cid = lax.axis_index("c")
    sid = lax.axis_index("s")
    wid = sid * 2 + cid
    row0 = wid * ROWS_PER_W

    pltpu.sync_copy(rden_hbm, rden_v)
    pltpu.sync_copy(bounds_hbm, bounds_v)

    zf = jnp.zeros((16,), jnp.float32)

    def zero_body(i, c):
        out_local[pl.ds(i * 16, 16)] = zf
        return c

    lax.fori_loop(0, ROWS_PER_W * D // 16, zero_body, 0)

    lo = plsc.load_gather(bounds_v, [jnp.full((16,), wid, jnp.int32)])[0]
    hi = plsc.load_gather(bounds_v, [jnp.full((16,), wid + 1, jnp.int32)])[0]
    st8 = lo & ~7
    nseg = (hi - st8 + (BUF - 1)) // BUF

    col0 = jnp.zeros((16,), jnp.int32)
    iota = lax.iota(jnp.int32, 16)

    def seg_body(g, c):
        start = pl.multiple_of(st8 + g * BUF, 8)
        pltpu.sync_copy(s_hbm.at[pl.ds(start, BUF)], s_v)
        pltpu.sync_copy(d_hbm.at[pl.ds(start, BUF)], d_v)
        pltpu.sync_copy(ex_hbm.at[pl.ds(start, BUF)], ex_v)
        nch = (jnp.minimum(hi, start + BUF) - start + 15) // 16

        def edge_body(i, c2):
            cps = []
            for q in range(4):
                off = i * 64 + q * 16
                s16 = s_v[pl.ds(off, 16)]
                cps.append(
                    pltpu.async_copy(h_hbm.at[s16], rows4[q], sems[q]))
            for q in range(4):
                off = i * 64 + q * 16
                gidx = start + off + iota
                d16 = d_v[pl.ds(off, 16)]
                ex16 = ex_v[pl.ds(off, 16)]
                valid = (gidx >= lo) & (gidx < hi)
                r16 = plsc.load_gather(rden_v, [col0, d16])
                alpha16 = jnp.where(valid, ex16 * r16, 0.0)
                dloc16 = jnp.where(valid, d16 - row0, 0) * D
                cps[q].wait()
                rows_v = rows4[q]
                for k in range(16):
                    av = jnp.full((16,), alpha16[k], jnp.float32)
                    base16 = jnp.full((16,), dloc16[k], jnp.int32) + iota
                    for j in range(8):
                        idx = base16 + j * 16
                        plsc.addupdate_scatter(
                            out_local, [idx],
                            av * rows_v[k, pl.ds(j * 16, 16)])
            return c2

        lax.fori_loop(0, (nch + 3) // 4, edge_body, 0)
        return c

    lax.fori_loop(0, nseg, seg_body, 0)
    pltpu.sync_copy(out_local, out_hbm.at[pl.ds(row0 * D, ROWS_PER_W * D)])


_s2 = functools.partial(
    pl.kernel,
    out_type=jax.ShapeDtypeStruct((NP * D,), jnp.float32),
    compiler_params=_SC_PARAMS,
    scratch_types=[pltpu.VMEM((1, NP), jnp.float32),
                   pltpu.VMEM((BUF,), jnp.int32),
                   pltpu.VMEM((BUF,), jnp.int32),
                   pltpu.VMEM((BUF,), jnp.float32),
                   pltpu.VMEM((16, D), jnp.float32),
                   pltpu.VMEM((16, D), jnp.float32),
                   pltpu.VMEM((16, D), jnp.float32),
                   pltpu.VMEM((16, D), jnp.float32),
                   pltpu.VMEM((ROWS_PER_W * D,), jnp.float32),
                   pltpu.VMEM((48,), jnp.int32),
                   pltpu.SemaphoreType.DMA,
                   pltpu.SemaphoreType.DMA,
                   pltpu.SemaphoreType.DMA,
                   pltpu.SemaphoreType.DMA],
)


def kernel(xq, xt, Wm, bm, W1, a_src1, a_dst1, b1, W2, a_src2, a_dst2, b2,
           W3, a_src3, a_dst3, b3, edge_index_q, edge_index_t):
    del edge_index_t
    f32 = jnp.float32
    src = edge_index_q[0].astype(jnp.int32)
    dst = edge_index_q[1].astype(jnp.int32)
    loop = jnp.arange(N, dtype=jnp.int32)
    padv = jnp.full((EP - E2,), N, jnp.int32)
    s2 = jnp.concatenate([src, loop, padv])
    d2 = jnp.concatenate([dst, loop, padv])
    perm = jnp.argsort(d2)
    d2 = d2[perm]
    s2 = s2[perm]
    bounds = jnp.searchsorted(
        d2, jnp.arange(NW + 1, dtype=jnp.int32) * ROWS_PER_W
    ).astype(jnp.int32)
    bounds = jnp.concatenate(
        [bounds, jnp.full((48 - NW - 1,), EP, jnp.int32)])
    s2p = jnp.concatenate([s2, jnp.zeros((BUF,), jnp.int32)])
    d2p = jnp.concatenate([d2, jnp.full((BUF,), N, jnp.int32)])

    xq_p = jnp.zeros((NP, D), f32).at[:N].set(xq.astype(f32))
    xt_p = jnp.zeros((NP, D), f32).at[:N].set(xt.astype(f32))
    bm2 = bm.reshape(1, D).astype(f32)

    def a2_of(a_s, a_d):
        z = jnp.zeros((D, 1), f32)
        return jnp.concatenate(
            [a_s.reshape(D, 1), a_d.reshape(D, 1), z, z], axis=1)

    layers = [(W1.astype(f32), a2_of(a_src1, a_dst1), b1.reshape(1, D)),
              (W2.astype(f32), a2_of(a_src2, a_dst2), b2.reshape(1, D)),
              (W3.astype(f32), a2_of(a_src3, a_dst3), b3.reshape(1, D))]

    mesh = plsc.VectorSubcoreMesh(**_MESH)
    s1 = _s1(_s1_body, mesh=mesh)
    s2k = _s2(_s2_body, mesh=mesh)

    pq = pt = None
    for li, (W, a2, b) in enumerate(layers):
        if li == 0:
            hq, ht, uvq, uvt = _t1a(xq_p, xt_p, Wm.astype(f32), bm2, W, a2)
        else:
            hq, ht, uvq, uvt = _t1b(pq, pt, layers[li - 1][2], W, a2)
        exq, denq = s1(uvq, s2, d2)
        dep = (denq[0, 0] * 0.0).astype(jnp.float32)
        ext, dent = s1(uvt + dep, s2, d2)
        rq, rt = _t2(denq, dent)
        zpad = jnp.zeros((BUF,), f32)
        exq_p = jnp.concatenate([exq, zpad])
        ext_p = jnp.concatenate([ext, zpad])
        pq = s2k(hq, rq + (rt[0, 0] * 0.0), s2p, d2p, exq_p, bounds)
        pt = s2k(ht, rt + (pq[0] * 0.0), s2p, d2p, ext_p, bounds)
        pq = pq.reshape(NP, D)
        pt = pt.reshape(NP, D)
    Xq, Xt = _t3(pq, pt, layers[2][2])
    return Xq[:N], Xt[:N]


# speedup vs baseline: 10.6023x; 1.0087x over previous
import functools

import jax
import jax.numpy as jnp
import numpy as np
from jax import lax
from jax.experimental import pallas as pl
from jax.experimental.pallas import tpu as pltpu
from jax.experimental.pallas import tpu_sc as plsc

N = 10000
NP = 10240
D = 128
E = 320000
E2 = E + N
NW = 32
ET = 10320
EP = NW * ET
CHUNKS = ET // 16
ROWS_PER_SUB = NP // 16

_MESH = dict(core_axis_name="c", subcore_axis_name="s",
             num_cores=2, num_subcores=16)


def _t1a_body(xq_ref, xt_ref, wm_ref, bm_ref, w_ref, a2_ref,
              hq_ref, ht_ref, uvq_ref, uvt_ref):
    wm = wm_ref[...]
    bm = bm_ref[...]
    w = w_ref[...]
    a2 = a2_ref[...]
    for x_ref, h_ref, uv_ref in ((xq_ref, hq_ref, uvq_ref),
                                 (xt_ref, ht_ref, uvt_ref)):
        x = jnp.dot(x_ref[...], wm, preferred_element_type=jnp.float32, precision=lax.Precision.HIGHEST) + bm
        h = jnp.dot(x, w, preferred_element_type=jnp.float32, precision=lax.Precision.HIGHEST)
        h_ref[...] = h
        uv_ref[...] = jnp.dot(h, a2, preferred_element_type=jnp.float32, precision=lax.Precision.HIGHEST)


def _elu(x):
    return jnp.where(x > 0, x, jnp.exp(jnp.minimum(x, 0.0)) - 1.0)


def _t1b_body(pq_ref, pt_ref, bprev_ref, w_ref, a2_ref,
              hq_ref, ht_ref, uvq_ref, uvt_ref):
    bprev = bprev_ref[...]
    w = w_ref[...]
    a2 = a2_ref[...]
    for p_ref, h_ref, uv_ref in ((pq_ref, hq_ref, uvq_ref),
                                 (pt_ref, ht_ref, uvt_ref)):
        x = _elu(p_ref[...] + bprev)
        h = jnp.dot(x, w, preferred_element_type=jnp.float32, precision=lax.Precision.HIGHEST)
        h_ref[...] = h
        uv_ref[...] = jnp.dot(h, a2, preferred_element_type=jnp.float32, precision=lax.Precision.HIGHEST)


def _t3_body(pq_ref, pt_ref, b_ref, xq_ref, xt_ref):
    b = b_ref[...]
    xq_ref[...] = _elu(pq_ref[...] + b)
    xt_ref[...] = _elu(pt_ref[...] + b)


def _t2_body(dq_ref, dt_ref, rq_ref, rt_ref):
    rq_ref[...] = 1.0 / (jnp.sum(dq_ref[...], axis=0, keepdims=True) + 1e-16)
    rt_ref[...] = 1.0 / (jnp.sum(dt_ref[...], axis=0, keepdims=True) + 1e-16)


_BLK = 256
_GRID = NP // _BLK


def _row_spec(d):
    return pl.BlockSpec((_BLK, d), lambda i: (i, 0))


def _full_spec(shape):
    return pl.BlockSpec(shape, lambda i: tuple(0 for _ in shape))


_t1a = pl.pallas_call(
    _t1a_body,
    grid=(_GRID,),
    in_specs=[_row_spec(D), _row_spec(D), _full_spec((D, D)),
              _full_spec((1, D)), _full_spec((D, D)), _full_spec((D, 4))],
    out_specs=[_row_spec(D), _row_spec(D), _row_spec(4), _row_spec(4)],
    out_shape=[jax.ShapeDtypeStruct((NP, D), jnp.float32),
               jax.ShapeDtypeStruct((NP, D), jnp.float32),
               jax.ShapeDtypeStruct((NP, 4), jnp.float32),
               jax.ShapeDtypeStruct((NP, 4), jnp.float32)],
)

_t1b = pl.pallas_call(
    _t1b_body,
    grid=(_GRID,),
    in_specs=[_row_spec(D), _row_spec(D), _full_spec((1, D)),
              _full_spec((D, D)), _full_spec((D, 4))],
    out_specs=[_row_spec(D), _row_spec(D), _row_spec(4), _row_spec(4)],
    out_shape=[jax.ShapeDtypeStruct((NP, D), jnp.float32),
               jax.ShapeDtypeStruct((NP, D), jnp.float32),
               jax.ShapeDtypeStruct((NP, 4), jnp.float32),
               jax.ShapeDtypeStruct((NP, 4), jnp.float32)],
)

_t3 = pl.pallas_call(
    _t3_body,
    grid=(_GRID,),
    in_specs=[_row_spec(D), _row_spec(D), _full_spec((1, D))],
    out_specs=[_row_spec(D), _row_spec(D)],
    out_shape=[jax.ShapeDtypeStruct((NP, D), jnp.float32),
               jax.ShapeDtypeStruct((NP, D), jnp.float32)],
)

_t2 = pl.pallas_call(
    _t2_body,
    in_specs=[pl.BlockSpec(memory_space=pltpu.VMEM),
              pl.BlockSpec(memory_space=pltpu.VMEM)],
    out_specs=[pl.BlockSpec(memory_space=pltpu.VMEM),
               pl.BlockSpec(memory_space=pltpu.VMEM)],
    out_shape=[jax.ShapeDtypeStruct((1, NP), jnp.float32),
               jax.ShapeDtypeStruct((1, NP), jnp.float32)],
)


def _s1_body(uv_hbm, s_hbm, d_hbm, ex_hbm, den_hbm,
             uv_v, s_v, d_v, ex_v, den_v, tmp_d, tmp_c):
    cid = lax.axis_index("c")
    sid = lax.axis_index("s")
    wid = sid * 2 + cid
    base = wid * ET
    pltpu.sync_copy(uv_hbm, uv_v)
    pltpu.sync_copy(s_hbm.at[pl.ds(base, ET)], s_v)
    pltpu.sync_copy(d_hbm.at[pl.ds(base, ET)], d_v)

    zf = jnp.zeros((16,), jnp.float32)

    def zero_body(i, c):
        den_v[pl.ds(i * 16, 16)] = zf
        return c

    lax.fori_loop(0, NP // 16, zero_body, 0)

    col0 = jnp.zeros((16,), jnp.int32)
    col1 = jnp.ones((16,), jnp.int32)
    iota = lax.iota(jnp.int32, 16)
    prev_i = jnp.maximum(iota - 1, 0)
    next_i = jnp.minimum(iota + 1, 15)
    first = iota == 0
    last = iota == 15

    def edge_body(i, c):
        off = i * 16
        s16 = s_v[pl.ds(off, 16)]
        d16 = d_v[pl.ds(off, 16)]
        u = plsc.load_gather(uv_v, [s16, col0])
        v = plsc.load_gather(uv_v, [d16, col1])
        e = u + v
        e = jnp.maximum(e, e * 0.2)
        ex = jnp.exp(e)
        ex_v[pl.ds(off, 16)] = ex
        dk, exs = plsc.sort_key_val(d16, ex)
        tmp_d[...] = dk
        c1 = plsc.cumsum(exs)
        tmp_c[...] = c1
        dprev = plsc.load_gather(tmp_d, [prev_i])
        dnext = plsc.load_gather(tmp_d, [next_i])
        cprev = jnp.where(first, 0.0, plsc.load_gather(tmp_c, [prev_i]))
        run_start = (dk != dprev) | first
        run_end = (dk != dnext) | last
        base = plsc.cummax(jnp.where(run_start, cprev, 0.0))
        plsc.addupdate_scatter(den_v, [dk], c1 - base, mask=run_end)
        return c

    lax.fori_loop(0, CHUNKS, edge_body, 0)

    pltpu.sync_copy(ex_v, ex_hbm.at[pl.ds(base, ET)])
    pltpu.sync_copy(den_v, den_hbm.at[wid])


_SC_PARAMS = pltpu.CompilerParams(needs_layout_passes=False,
                                  use_tc_tiling_on_sc=False)

_s1 = functools.partial(
    pl.kernel,
    out_type=[jax.ShapeDtypeStruct((EP,), jnp.float32),
              jax.ShapeDtypeStruct((NW, NP), jnp.float32)],
    compiler_params=_SC_PARAMS,
    scratch_types=[pltpu.VMEM((NP, 4), jnp.float32),
                   pltpu.VMEM((ET,), jnp.int32),
                   pltpu.VMEM((ET,), jnp.int32),
                   pltpu.VMEM((ET,), jnp.float32),
                   pltpu.VMEM((NP,), jnp.float32),
                   pltpu.VMEM((16,), jnp.int32),
                   pltpu.VMEM((16,), jnp.float32)],
)


ROWS_PER_W = NP // NW
BUF = 6144
EPP = EP + BUF

_GDN = lax.GatherDimensionNumbers(
    offset_dims=(), collapsed_slice_dims=(0,), start_index_map=(0,))
def _splat(vec, k):
    return lax.gather(vec, jnp.full((16, 1), k, jnp.int32), _GDN, (1,),
                      mode=lax.GatherScatterMode.PROMISE_IN_BOUNDS)


def _s2_body(h_hbm, rden_hbm, s_hbm, d_hbm, ex_hbm, bounds_hbm, out_hbm,
             rden_v, s_v, d_v, ex_v, r0, r1, r2, r3,
             out_local, bounds_v, m0, m1, m2, m3):
    rows4 = (r0, r1, r2, r3)
    sems = (m0, m1, m2, m3)
    cid = lax.axis_index("c")
    sid = lax.axis_index("s")
    wid = sid * 2 + cid
    row0 = wid * ROWS_PER_W

    pltpu.sync_copy(rden_hbm, rden_v)
    pltpu.sync_copy(bounds_hbm, bounds_v)

    zf = jnp.zeros((16,), jnp.float32)

    def zero_body(i, c):
        out_local[pl.ds(i * 16, 16)] = zf
        return c

    lax.fori_loop(0, ROWS_PER_W * D // 16, zero_body, 0)

    lo = plsc.load_gather(bounds_v, [jnp.full((16,), wid, jnp.int32)])[0]
    hi = plsc.load_gather(bounds_v, [jnp.full((16,), wid + 1, jnp.int32)])[0]
    st8 = lo & ~7
    nseg = (hi - st8 + (BUF - 1)) // BUF

    col0 = jnp.zeros((16,), jnp.int32)
    iota = lax.iota(jnp.int32, 16)

    def seg_body(g, c):
        start = pl.multiple_of(st8 + g * BUF, 8)
        pltpu.sync_copy(s_hbm.at[pl.ds(start, BUF)], s_v)
        pltpu.sync_copy(d_hbm.at[pl.ds(start, BUF)], d_v)
        pltpu.sync_copy(ex_hbm.at[pl.ds(start, BUF)], ex_v)
        nch = (jnp.minimum(hi, start + BUF) - start + 15) // 16

        def edge_body(i, c2):
            cps = []
            for q in range(4):
                off = i * 64 + q * 16
                s16 = s_v[pl.ds(off, 16)]
                cps.append(
                    pltpu.async_copy(h_hbm.at[s16], rows4[q], sems[q]))
            for q in range(4):
                off = i * 64 + q * 16
                gidx = start + off + iota
                d16 = d_v[pl.ds(off, 16)]
                ex16 = ex_v[pl.ds(off, 16)]
                valid = (gidx >= lo) & (gidx < hi)
                r16 = plsc.load_gather(rden_v, [col0, d16])
                alpha16 = jnp.where(valid, ex16 * r16, 0.0)
                dloc16 = jnp.where(valid, d16 - row0, 0) * D
                cps[q].wait()
                rows_v = rows4[q]
                for k in range(16):
                    av = _splat(alpha16, k)
                    base16 = _splat(dloc16, k) + iota
                    for j in range(8):
                        idx = base16 + j * 16
                        plsc.addupdate_scatter(
                            out_local, [idx],
                            av * rows_v[k, pl.ds(j * 16, 16)])
            return c2

        lax.fori_loop(0, (nch + 3) // 4, edge_body, 0)
        return c

    lax.fori_loop(0, nseg, seg_body, 0)
    pltpu.sync_copy(out_local, out_hbm.at[pl.ds(row0 * D, ROWS_PER_W * D)])


_s2 = functools.partial(
    pl.kernel,
    out_type=jax.ShapeDtypeStruct((NP * D,), jnp.float32),
    compiler_params=_SC_PARAMS,
    scratch_types=[pltpu.VMEM((1, NP), jnp.float32),
                   pltpu.VMEM((BUF,), jnp.int32),
                   pltpu.VMEM((BUF,), jnp.int32),
                   pltpu.VMEM((BUF,), jnp.float32),
                   pltpu.VMEM((16, D), jnp.float32),
                   pltpu.VMEM((16, D), jnp.float32),
                   pltpu.VMEM((16, D), jnp.float32),
                   pltpu.VMEM((16, D), jnp.float32),
                   pltpu.VMEM((ROWS_PER_W * D,), jnp.float32),
                   pltpu.VMEM((48,), jnp.int32),
                   pltpu.SemaphoreType.DMA,
                   pltpu.SemaphoreType.DMA,
                   pltpu.SemaphoreType.DMA,
                   pltpu.SemaphoreType.DMA],
)


def kernel(xq, xt, Wm, bm, W1, a_src1, a_dst1, b1, W2, a_src2, a_dst2, b2,
           W3, a_src3, a_dst3, b3, edge_index_q, edge_index_t):
    del edge_index_t
    f32 = jnp.float32
    src = edge_index_q[0].astype(jnp.int32)
    dst = edge_index_q[1].astype(jnp.int32)
    loop = jnp.arange(N, dtype=jnp.int32)
    padv = jnp.full((EP - E2,), N, jnp.int32)
    s2 = jnp.concatenate([src, loop, padv])
    d2 = jnp.concatenate([dst, loop, padv])
    perm = jnp.argsort(d2)
    d2 = d2[perm]
    s2 = s2[perm]
    bounds = jnp.searchsorted(
        d2, jnp.arange(NW + 1, dtype=jnp.int32) * ROWS_PER_W
    ).astype(jnp.int32)
    bounds = jnp.concatenate(
        [bounds, jnp.full((48 - NW - 1,), EP, jnp.int32)])
    s2p = jnp.concatenate([s2, jnp.zeros((BUF,), jnp.int32)])
    d2p = jnp.concatenate([d2, jnp.full((BUF,), N, jnp.int32)])

    xq_p = jnp.zeros((NP, D), f32).at[:N].set(xq.astype(f32))
    xt_p = jnp.zeros((NP, D), f32).at[:N].set(xt.astype(f32))
    bm2 = bm.reshape(1, D).astype(f32)

    def a2_of(a_s, a_d):
        z = jnp.zeros((D, 1), f32)
        return jnp.concatenate(
            [a_s.reshape(D, 1), a_d.reshape(D, 1), z, z], axis=1)

    layers = [(W1.astype(f32), a2_of(a_src1, a_dst1), b1.reshape(1, D)),
              (W2.astype(f32), a2_of(a_src2, a_dst2), b2.reshape(1, D)),
              (W3.astype(f32), a2_of(a_src3, a_dst3), b3.reshape(1, D))]

    mesh = plsc.VectorSubcoreMesh(**_MESH)
    s1 = _s1(_s1_body, mesh=mesh)
    s2k = _s2(_s2_body, mesh=mesh)

    pq = pt = None
    for li, (W, a2, b) in enumerate(layers):
        if li == 0:
            hq, ht, uvq, uvt = _t1a(xq_p, xt_p, Wm.astype(f32), bm2, W, a2)
        else:
            hq, ht, uvq, uvt = _t1b(pq, pt, layers[li - 1][2], W, a2)
        exq, denq = s1(uvq, s2, d2)
        dep = (denq[0, 0] * 0.0).astype(jnp.float32)
        ext, dent = s1(uvt + dep, s2, d2)
        rq, rt = _t2(denq, dent)
        zpad = jnp.zeros((BUF,), f32)
        exq_p = jnp.concatenate([exq, zpad])
        ext_p = jnp.concatenate([ext, zpad])
        pq = s2k(hq, rq + (rt[0, 0] * 0.0), s2p, d2p, exq_p, bounds)
        pt = s2k(ht, rt + (pq[0] * 0.0), s2p, d2p, ext_p, bounds)
        pq = pq.reshape(NP, D)
        pt = pt.reshape(NP, D)
    Xq, Xt = _t3(pq, pt, layers[2][2])
    return Xq[:N], Xt[:N]
